# baseline (speedup 1.0000x reference)
import functools

import jax
import jax.numpy as jnp
from jax import lax
from jax.experimental import pallas as pl
from jax.experimental.pallas import tpu as pltpu
from jax.experimental.pallas import tpu_sc as plsc

_NW = 32
_BBLK = 128
_NBUF = 5
_NSLOT = 2 * _NBUF


@functools.partial(jax.jit, static_argnums=(2, 3, 4))
def _emb_lookup_t(idx_t, table, nb, hist, d):
    mesh = plsc.VectorSubcoreMesh(core_axis_name="c", subcore_axis_name="s")

    @functools.partial(
        pl.kernel,
        out_type=jax.ShapeDtypeStruct((nb, hist, d), jnp.float32),
        mesh=mesh,
        scratch_types=[
            pltpu.VMEM((hist, _BBLK), jnp.int32),
            pltpu.VMEM((_NSLOT, _BBLK, d), jnp.float32),
            [pltpu.SemaphoreType.DMA] * _NSLOT,
            [pltpu.SemaphoreType.DMA] * _NSLOT,
        ],
        compiler_params=pltpu.CompilerParams(use_tc_tiling_on_sc=False),
    )
    def emb(table_hbm, idx_hbm, out_hbm, idx_v, rows_v, gsems, wsems):
        wid = lax.axis_index("s") * 2 + lax.axis_index("c")
        b0 = wid * _BBLK
        pltpu.sync_copy(idx_hbm.at[:, pl.ds(b0, _BBLK)], idx_v)

        def start_gather(h, b):
            pltpu.async_copy(table_hbm.at[idx_v.at[h]], rows_v.at[b], gsems[b])

        def wait_gather(h, b):
            pltpu.make_async_copy(
                table_hbm.at[idx_v.at[h]], rows_v.at[b], gsems[b]
            ).wait()

        def write(h, b):
            return pltpu.make_async_copy(
                rows_v.at[b],
                out_hbm.at[pl.ds(b0, _BBLK), h, :],
                wsems[b],
            )

        for h in range(min(_NBUF, hist)):
            start_gather(h, h % _NSLOT)

        for h in range(hist):
            bb = h % _NSLOT
            wait_gather(h, bb)
            write(h, bb).start()
            hn = h + _NBUF
            if hn < hist:
                bb2 = hn % _NSLOT
                if h >= _NBUF:
                    write(h - _NBUF, bb2).wait()
                start_gather(hn, bb2)

        for h in range(max(0, hist - _NSLOT), hist):
            write(h, h % _NSLOT).wait()

    return emb(table, idx_t)


def kernel(indices, table):
    nb, hist = indices.shape
    _, d = table.shape
    assert nb % (_NW * _BBLK) == 0 or nb == _NW * _BBLK
    return _emb_lookup_t(indices.T, table, nb, hist, d)


# speedup vs baseline: 1.0568x; 1.0568x over previous
import functools

import jax
import jax.numpy as jnp
from jax import lax
from jax.experimental import pallas as pl
from jax.experimental.pallas import tpu as pltpu
from jax.experimental.pallas import tpu_sc as plsc

_NW = 32
_BBLK = 128
_NBUF = 6
_NSLOT = 2 * _NBUF


@functools.partial(jax.jit, static_argnums=(2, 3, 4))
def _emb_lookup_t(idx_t, table, nb, hist, d):
    mesh = plsc.VectorSubcoreMesh(core_axis_name="c", subcore_axis_name="s")

    @functools.partial(
        pl.kernel,
        out_type=jax.ShapeDtypeStruct((hist, nb, d), jnp.float32),
        mesh=mesh,
        scratch_types=[
            pltpu.VMEM((hist, _BBLK), jnp.int32),
            pltpu.VMEM((_NSLOT, _BBLK, d), jnp.float32),
            [pltpu.SemaphoreType.DMA] * _NSLOT,
            [pltpu.SemaphoreType.DMA] * _NSLOT,
        ],
        compiler_params=pltpu.CompilerParams(use_tc_tiling_on_sc=False),
    )
    def emb(table_hbm, idx_hbm, out_hbm, idx_v, rows_v, gsems, wsems):
        wid = lax.axis_index("s") * 2 + lax.axis_index("c")
        b0 = wid * _BBLK
        pltpu.sync_copy(idx_hbm.at[:, pl.ds(b0, _BBLK)], idx_v)

        def start_gather(h, b):
            pltpu.async_copy(table_hbm.at[idx_v.at[h]], rows_v.at[b], gsems[b])

        def wait_gather(h, b):
            pltpu.make_async_copy(
                table_hbm.at[idx_v.at[h]], rows_v.at[b], gsems[b]
            ).wait()

        def write(h, b):
            return pltpu.make_async_copy(
                rows_v.at[b],
                out_hbm.at[h, pl.ds(b0, _BBLK), :],
                wsems[b],
            )

        for h in range(min(_NBUF, hist)):
            start_gather(h, h % _NSLOT)

        for h in range(hist):
            bb = h % _NSLOT
            wait_gather(h, bb)
            write(h, bb).start()
            hn = h + _NBUF
            if hn < hist:
                bb2 = hn % _NSLOT
                if h >= _NBUF:
                    write(h - _NBUF, bb2).wait()
                start_gather(hn, bb2)

        for h in range(max(0, hist - _NSLOT), hist):
            write(h, h % _NSLOT).wait()

    return emb(table, idx_t)


def kernel(indices, table):
    nb, hist = indices.shape
    _, d = table.shape
    assert nb % (_NW * _BBLK) == 0 or nb == _NW * _BBLK
    out_t = _emb_lookup_t(indices.T, table, nb, hist, d)
    return jnp.transpose(out_t, (1, 0, 2))
